# baseline (speedup 1.0000x reference)
import functools

import jax
import jax.numpy as jnp
from jax.experimental import pallas as pl
from jax.experimental.pallas import tpu as pltpu

INPUT_DIM = 768
NUM_PRIMES = 8192
K = 8
AMP_DIM = 4
AK = K * AMP_DIM

ROW_BLOCK = 1024
PRIME_TILE = 2048
NUM_TILES = NUM_PRIMES // PRIME_TILE
SLOT = 128


STRIP = 8


def _fused(hs_ref, sw_ref, sb_ref, aw_ref, ab_ref,
           idx_ref, amp_ref, sc_ref, cv_ref, ci_ref):
    j = pl.program_id(1)
    hs = hs_ref[...]
    sc_ref[...] = jax.lax.dot_general(
        hs, sw_ref[...], (((1,), (1,)), ((), ())),
        preferred_element_type=jnp.float32) + sb_ref[...]

    lane = jax.lax.broadcasted_iota(jnp.int32, (STRIP, PRIME_TILE), 1)
    base = j * PRIME_TILE

    def strip_body(s, _):
        x = sc_ref[pl.ds(s * STRIP, STRIP), :]
        vals, idxs = [], []
        for _ in range(K):
            m = jnp.max(x, axis=1, keepdims=True)
            pos = jnp.argmax(x, axis=1).astype(jnp.int32).reshape(STRIP, 1)
            vals.append(m)
            idxs.append(pos + base)
            x = jnp.where(lane == pos, -jnp.inf, x)
        cv_ref[j, pl.ds(s * STRIP, STRIP), :] = jnp.concatenate(vals, axis=1)
        ci_ref[j, pl.ds(s * STRIP, STRIP), :] = jnp.concatenate(idxs, axis=1)
        return 0

    jax.lax.fori_loop(0, ROW_BLOCK // STRIP, strip_body, 0)

    @pl.when(j == NUM_TILES - 1)
    def _merge():
        nc = NUM_TILES * K
        cv = jnp.concatenate([cv_ref[t] for t in range(NUM_TILES)], axis=1)
        ci = jnp.concatenate([ci_ref[t] for t in range(NUM_TILES)], axis=1)
        slot = jax.lax.broadcasted_iota(jnp.int32, (ROW_BLOCK, nc), 1)
        x2 = cv
        fv, fi = [], []
        for _ in range(K):
            m = jnp.max(x2, axis=1, keepdims=True)
            pos = jnp.argmax(x2, axis=1).astype(jnp.int32).reshape(ROW_BLOCK, 1)
            sel = slot == pos
            fv.append(m)
            fi.append(jnp.sum(jnp.where(sel, ci, 0), axis=1, keepdims=True))
            x2 = jnp.where(sel, -jnp.inf, x2)
        topv = jnp.concatenate(fv, axis=1)
        idx_ref[...] = jnp.concatenate(fi, axis=1)

        w = jnp.exp(topv - topv[:, :1])
        w = w / jnp.sum(w, axis=1, keepdims=True)

        amps = jax.lax.dot_general(
            hs, aw_ref[...], (((1,), (1,)), ((), ())),
            preferred_element_type=jnp.float32) + ab_ref[...]

        r8 = jax.lax.broadcasted_iota(jnp.int32, (K, AK), 0)
        c32 = jax.lax.broadcasted_iota(jnp.int32, (K, AK), 1)
        expand = (c32 // AMP_DIM == r8).astype(jnp.float32)
        w32 = jax.lax.dot_general(
            w, expand, (((1,), (0,)), ((), ())),
            preferred_element_type=jnp.float32)
        wa = amps * w32
        g = wa * wa
        p = jax.lax.broadcasted_iota(jnp.int32, (AK, AK), 0)
        q = jax.lax.broadcasted_iota(jnp.int32, (AK, AK), 1)
        gsum = (p // AMP_DIM == q // AMP_DIM).astype(jnp.float32)
        n2 = jax.lax.dot_general(
            g, gsum, (((1,), (0,)), ((), ())),
            preferred_element_type=jnp.float32)
        amp_ref[...] = wa / jnp.maximum(jnp.sqrt(n2), 1e-12)


@functools.partial(jax.jit, static_argnames=())
def kernel(hidden_states, score_w, score_b, amp_w, amp_b):
    batch, seq, d = hidden_states.shape
    rows = batch * seq
    hs2 = hidden_states.reshape(rows, d)
    sb2 = score_b.reshape(1, NUM_PRIMES)
    ab2 = amp_b.reshape(1, AK)
    nr = rows // ROW_BLOCK

    idx_out, amp_out = pl.pallas_call(
        _fused,
        grid=(nr, NUM_TILES),
        in_specs=[
            pl.BlockSpec((ROW_BLOCK, d), lambda i, j: (i, 0)),
            pl.BlockSpec((PRIME_TILE, d), lambda i, j: (j, 0)),
            pl.BlockSpec((1, PRIME_TILE), lambda i, j: (0, j)),
            pl.BlockSpec((AK, d), lambda i, j: (0, 0)),
            pl.BlockSpec((1, AK), lambda i, j: (0, 0)),
        ],
        out_specs=[
            pl.BlockSpec((ROW_BLOCK, K), lambda i, j: (i, 0)),
            pl.BlockSpec((ROW_BLOCK, AK), lambda i, j: (i, 0)),
        ],
        out_shape=[
            jax.ShapeDtypeStruct((rows, K), jnp.int32),
            jax.ShapeDtypeStruct((rows, AK), jnp.float32),
        ],
        scratch_shapes=[
            pltpu.VMEM((ROW_BLOCK, PRIME_TILE), jnp.float32),
            pltpu.VMEM((NUM_TILES, ROW_BLOCK, K), jnp.float32),
            pltpu.VMEM((NUM_TILES, ROW_BLOCK, K), jnp.int32),
        ],
        compiler_params=pltpu.CompilerParams(
            dimension_semantics=("parallel", "arbitrary")),
    )(hs2, score_w, sb2, amp_w, ab2)

    topk_indices = idx_out.reshape(batch, seq, K)
    amps = amp_out.reshape(batch, seq, K, AMP_DIM)
    return (topk_indices, amps)


# speedup vs baseline: 6.8886x; 6.8886x over previous
import functools

import jax
import jax.numpy as jnp
from jax.experimental import pallas as pl
from jax.experimental.pallas import tpu as pltpu

INPUT_DIM = 768
NUM_PRIMES = 8192
K = 8
AMP_DIM = 4
AK = K * AMP_DIM

ROW_BLOCK = 1024
PRIME_TILE = 2048
NUM_TILES = NUM_PRIMES // PRIME_TILE
SLOT = 128


def _fused(hs_ref, sw_ref, sb_ref, aw_ref, ab_ref,
           idx_ref, amp_ref, cv_ref, ci_ref):
    j = pl.program_id(1)
    hs = hs_ref[...]
    x = jax.lax.dot_general(
        hs, sw_ref[...], (((1,), (1,)), ((), ())),
        preferred_element_type=jnp.float32) + sb_ref[...]

    lane = jax.lax.broadcasted_iota(jnp.int32, (ROW_BLOCK, PRIME_TILE), 1)
    base = j * PRIME_TILE
    vals, idxs = [], []
    for _ in range(K):
        m = jnp.max(x, axis=1, keepdims=True)
        pos = jnp.argmax(x, axis=1).astype(jnp.int32).reshape(ROW_BLOCK, 1)
        vals.append(m)
        idxs.append(pos + base)
        x = jnp.where(lane == pos, -jnp.inf, x)
    cv_ref[j] = jnp.concatenate(vals, axis=1)
    ci_ref[j] = jnp.concatenate(idxs, axis=1)

    @pl.when(j == NUM_TILES - 1)
    def _merge():
        nc = NUM_TILES * K
        cv = jnp.concatenate([cv_ref[t] for t in range(NUM_TILES)], axis=1)
        ci = jnp.concatenate([ci_ref[t] for t in range(NUM_TILES)], axis=1)
        slot = jax.lax.broadcasted_iota(jnp.int32, (ROW_BLOCK, nc), 1)
        x2 = cv
        fv, fi = [], []
        for _ in range(K):
            m = jnp.max(x2, axis=1, keepdims=True)
            pos = jnp.argmax(x2, axis=1).astype(jnp.int32).reshape(ROW_BLOCK, 1)
            sel = slot == pos
            fv.append(m)
            fi.append(jnp.sum(jnp.where(sel, ci, 0), axis=1, keepdims=True))
            x2 = jnp.where(sel, -jnp.inf, x2)
        topv = jnp.concatenate(fv, axis=1)
        idx_ref[...] = jnp.concatenate(fi, axis=1)

        w = jnp.exp(topv - topv[:, :1])
        w = w / jnp.sum(w, axis=1, keepdims=True)

        amps = jax.lax.dot_general(
            hs, aw_ref[...], (((1,), (1,)), ((), ())),
            preferred_element_type=jnp.float32) + ab_ref[...]

        r8 = jax.lax.broadcasted_iota(jnp.int32, (K, AK), 0)
        c32 = jax.lax.broadcasted_iota(jnp.int32, (K, AK), 1)
        expand = (c32 // AMP_DIM == r8).astype(jnp.float32)
        w32 = jax.lax.dot_general(
            w, expand, (((1,), (0,)), ((), ())),
            preferred_element_type=jnp.float32)
        wa = amps * w32
        g = wa * wa
        p = jax.lax.broadcasted_iota(jnp.int32, (AK, AK), 0)
        q = jax.lax.broadcasted_iota(jnp.int32, (AK, AK), 1)
        gsum = (p // AMP_DIM == q // AMP_DIM).astype(jnp.float32)
        n2 = jax.lax.dot_general(
            g, gsum, (((1,), (0,)), ((), ())),
            preferred_element_type=jnp.float32)
        amp_ref[...] = wa / jnp.maximum(jnp.sqrt(n2), 1e-12)


@functools.partial(jax.jit, static_argnames=())
def kernel(hidden_states, score_w, score_b, amp_w, amp_b):
    batch, seq, d = hidden_states.shape
    rows = batch * seq
    hs2 = hidden_states.reshape(rows, d)
    sb2 = score_b.reshape(1, NUM_PRIMES)
    ab2 = amp_b.reshape(1, AK)
    nr = rows // ROW_BLOCK

    idx_out, amp_out = pl.pallas_call(
        _fused,
        grid=(nr, NUM_TILES),
        in_specs=[
            pl.BlockSpec((ROW_BLOCK, d), lambda i, j: (i, 0)),
            pl.BlockSpec((PRIME_TILE, d), lambda i, j: (j, 0)),
            pl.BlockSpec((1, PRIME_TILE), lambda i, j: (0, j)),
            pl.BlockSpec((AK, d), lambda i, j: (0, 0)),
            pl.BlockSpec((1, AK), lambda i, j: (0, 0)),
        ],
        out_specs=[
            pl.BlockSpec((ROW_BLOCK, K), lambda i, j: (i, 0)),
            pl.BlockSpec((ROW_BLOCK, AK), lambda i, j: (i, 0)),
        ],
        out_shape=[
            jax.ShapeDtypeStruct((rows, K), jnp.int32),
            jax.ShapeDtypeStruct((rows, AK), jnp.float32),
        ],
        scratch_shapes=[
            pltpu.VMEM((NUM_TILES, ROW_BLOCK, K), jnp.float32),
            pltpu.VMEM((NUM_TILES, ROW_BLOCK, K), jnp.int32),
        ],
        compiler_params=pltpu.CompilerParams(
            dimension_semantics=("parallel", "arbitrary")),
    )(hs2, score_w, sb2, amp_w, ab2)

    topk_indices = idx_out.reshape(batch, seq, K)
    amps = amp_out.reshape(batch, seq, K, AMP_DIM)
    return (topk_indices, amps)


# speedup vs baseline: 7.7212x; 1.1209x over previous
import functools

import jax
import jax.numpy as jnp
from jax.experimental import pallas as pl
from jax.experimental.pallas import tpu as pltpu

INPUT_DIM = 768
NUM_PRIMES = 8192
K = 8
AMP_DIM = 4
AK = K * AMP_DIM

ROW_BLOCK = 1024
PRIME_TILE = 2048
NUM_TILES = NUM_PRIMES // PRIME_TILE
SLOT = 128


def _fused(hs_ref, sw_ref, sb_ref, aw_ref, ab_ref,
           idx_ref, amp_ref, cv_ref, ci_ref):
    j = pl.program_id(1)
    hs = hs_ref[...]
    x = jax.lax.dot_general(
        hs, sw_ref[...], (((1,), (1,)), ((), ())),
        preferred_element_type=jnp.float32) + sb_ref[...]

    lane = jax.lax.broadcasted_iota(jnp.int32, (ROW_BLOCK, PRIME_TILE), 1)
    base = j * PRIME_TILE
    vals, idxs = [], []
    for _ in range(K):
        m = jnp.max(x, axis=1, keepdims=True)
        hit = x == m
        pos = jnp.min(jnp.where(hit, lane, PRIME_TILE), axis=1, keepdims=True)
        vals.append(m)
        idxs.append(pos + base)
        x = jnp.where(lane == pos, -jnp.inf, x)
    vpad = jnp.full((ROW_BLOCK, SLOT - K), -jnp.inf, dtype=jnp.float32)
    ipad = jnp.zeros((ROW_BLOCK, SLOT - K), dtype=jnp.int32)
    cv_ref[:, pl.ds(j * SLOT, SLOT)] = jnp.concatenate(vals + [vpad], axis=1)
    ci_ref[:, pl.ds(j * SLOT, SLOT)] = jnp.concatenate(idxs + [ipad], axis=1)

    @pl.when(j == NUM_TILES - 1)
    def _merge():
        nc = NUM_TILES * SLOT
        cv = cv_ref[...]
        ci = ci_ref[...]
        slot = jax.lax.broadcasted_iota(jnp.int32, (ROW_BLOCK, nc), 1)
        x2 = cv
        fv, fi = [], []
        for _ in range(K):
            m = jnp.max(x2, axis=1, keepdims=True)
            pos = jnp.min(jnp.where(x2 == m, slot, nc), axis=1, keepdims=True)
            sel = slot == pos
            fv.append(m)
            fi.append(jnp.sum(jnp.where(sel, ci, 0), axis=1, keepdims=True))
            x2 = jnp.where(sel, -jnp.inf, x2)
        topv = jnp.concatenate(fv, axis=1)
        idx_ref[...] = jnp.concatenate(fi, axis=1)

        w = jnp.exp(topv - topv[:, :1])
        w = w / jnp.sum(w, axis=1, keepdims=True)

        amps = jax.lax.dot_general(
            hs, aw_ref[...], (((1,), (1,)), ((), ())),
            preferred_element_type=jnp.float32) + ab_ref[...]

        r8 = jax.lax.broadcasted_iota(jnp.int32, (K, AK), 0)
        c32 = jax.lax.broadcasted_iota(jnp.int32, (K, AK), 1)
        expand = (c32 // AMP_DIM == r8).astype(jnp.float32)
        w32 = jax.lax.dot_general(
            w, expand, (((1,), (0,)), ((), ())),
            preferred_element_type=jnp.float32)
        wa = amps * w32
        g = wa * wa
        p = jax.lax.broadcasted_iota(jnp.int32, (AK, AK), 0)
        q = jax.lax.broadcasted_iota(jnp.int32, (AK, AK), 1)
        gsum = (p // AMP_DIM == q // AMP_DIM).astype(jnp.float32)
        n2 = jax.lax.dot_general(
            g, gsum, (((1,), (0,)), ((), ())),
            preferred_element_type=jnp.float32)
        amp_ref[...] = wa / jnp.maximum(jnp.sqrt(n2), 1e-12)


@functools.partial(jax.jit, static_argnames=())
def kernel(hidden_states, score_w, score_b, amp_w, amp_b):
    batch, seq, d = hidden_states.shape
    rows = batch * seq
    hs2 = hidden_states.reshape(rows, d)
    sb2 = score_b.reshape(1, NUM_PRIMES)
    ab2 = amp_b.reshape(1, AK)
    nr = rows // ROW_BLOCK

    idx_out, amp_out = pl.pallas_call(
        _fused,
        grid=(nr, NUM_TILES),
        in_specs=[
            pl.BlockSpec((ROW_BLOCK, d), lambda i, j: (i, 0)),
            pl.BlockSpec((PRIME_TILE, d), lambda i, j: (j, 0)),
            pl.BlockSpec((1, PRIME_TILE), lambda i, j: (0, j)),
            pl.BlockSpec((AK, d), lambda i, j: (0, 0)),
            pl.BlockSpec((1, AK), lambda i, j: (0, 0)),
        ],
        out_specs=[
            pl.BlockSpec((ROW_BLOCK, K), lambda i, j: (i, 0)),
            pl.BlockSpec((ROW_BLOCK, AK), lambda i, j: (i, 0)),
        ],
        out_shape=[
            jax.ShapeDtypeStruct((rows, K), jnp.int32),
            jax.ShapeDtypeStruct((rows, AK), jnp.float32),
        ],
        scratch_shapes=[
            pltpu.VMEM((ROW_BLOCK, NUM_TILES * SLOT), jnp.float32),
            pltpu.VMEM((ROW_BLOCK, NUM_TILES * SLOT), jnp.int32),
        ],
        compiler_params=pltpu.CompilerParams(
            dimension_semantics=("parallel", "arbitrary")),
    )(hs2, score_w, sb2, amp_w, ab2)

    topk_indices = idx_out.reshape(batch, seq, K)
    amps = amp_out.reshape(batch, seq, K, AMP_DIM)
    return (topk_indices, amps)
